# baseline (speedup 1.0000x reference)
import jax
import jax.numpy as jnp
from jax.experimental import pallas as pl
from jax.experimental.pallas import tpu as pltpu

_NUM_CODES = 1024
_CODE_DIM = 64
_COMMITMENT_COST = 0.25
_BLOCK_ROWS = 2


def _vq_block_kernel(z_ref, emb_ref, zq_ref, idx_ref, loss_ref):
    blk = _BLOCK_ROWS * 1024
    z = z_ref[...].reshape(blk, _CODE_DIM)
    emb = emb_ref[...]
    z_sq = jnp.sum(z * z, axis=1, keepdims=True)
    e_sq = jnp.sum(emb * emb, axis=1)
    neg2_emb = -2.0 * emb
    scores2 = jax.lax.dot_general(
        z, neg2_emb, (((1,), (1,)), ((), ())),
        preferred_element_type=jnp.float32)
    d = (z_sq + scores2) + e_sq[None, :]
    dmin = jnp.min(d, axis=1)
    iota = jax.lax.broadcasted_iota(jnp.int32, (1, _NUM_CODES), 1).astype(
        jnp.float32)
    idx = jnp.min(
        jnp.where(d == dmin[:, None], iota, float(_NUM_CODES)), axis=1
    ).astype(jnp.int32)

    zq_ref[...] = z_ref[...]
    idx_ref[0, :, :] = idx.reshape(_BLOCK_ROWS, 1024)
    loss_ref[0, 0, 0] = jnp.sum(dmin)


def kernel(z, embedding):
    rows = z.shape[0]
    grid = rows // _BLOCK_ROWS

    zq, idx3, loss_sum = pl.pallas_call(
        _vq_block_kernel,
        grid=(grid,),
        in_specs=[
            pl.BlockSpec((_BLOCK_ROWS, 1024, _CODE_DIM), lambda i: (i, 0, 0)),
            pl.BlockSpec((_NUM_CODES, _CODE_DIM), lambda i: (0, 0)),
        ],
        out_specs=[
            pl.BlockSpec((_BLOCK_ROWS, 1024, _CODE_DIM), lambda i: (i, 0, 0)),
            pl.BlockSpec((1, _BLOCK_ROWS, 1024), lambda i: (i, 0, 0)),
            pl.BlockSpec((1, 1, 1), lambda i: (i, 0, 0),
                         memory_space=pltpu.SMEM),
        ],
        out_shape=[
            jax.ShapeDtypeStruct(z.shape, jnp.float32),
            jax.ShapeDtypeStruct((grid, _BLOCK_ROWS, 1024), jnp.int32),
            jax.ShapeDtypeStruct((grid, 1, 1), jnp.float32),
        ],
        compiler_params=pltpu.CompilerParams(
            dimension_semantics=("parallel",)),
    )(z, embedding)

    indices = idx3.reshape(z.shape[:-1])
    vq_loss = _COMMITMENT_COST * jnp.sum(loss_sum) / (rows * 1024 * _CODE_DIM)
    return (zq, indices, vq_loss)


# speedup vs baseline: 1.0161x; 1.0161x over previous
import jax
import jax.numpy as jnp
from jax.experimental import pallas as pl
from jax.experimental.pallas import tpu as pltpu

_NUM_CODES = 1024
_CODE_DIM = 64
_COMMITMENT_COST = 0.25
_BLOCK_ROWS = 4


def _vq_block_kernel(z_ref, emb_ref, zq_ref, idx_ref, loss_ref):
    blk = _BLOCK_ROWS * 1024
    z = z_ref[...].reshape(blk, _CODE_DIM)
    emb = emb_ref[...]
    z_sq = jnp.sum(z * z, axis=1, keepdims=True)
    e_sq = jnp.sum(emb * emb, axis=1)
    neg2_emb = -2.0 * emb
    scores2 = jax.lax.dot_general(
        z, neg2_emb, (((1,), (1,)), ((), ())),
        preferred_element_type=jnp.float32)
    d = (z_sq + scores2) + e_sq[None, :]
    dmin = jnp.min(d, axis=1)
    iota = jax.lax.broadcasted_iota(jnp.int32, (1, _NUM_CODES), 1).astype(
        jnp.float32)
    idx = jnp.min(
        jnp.where(d == dmin[:, None], iota, float(_NUM_CODES)), axis=1
    ).astype(jnp.int32)

    zq_ref[...] = z_ref[...]
    idx_ref[0, :, :] = idx.reshape(_BLOCK_ROWS, 1024)
    loss_ref[0, 0, 0] = jnp.sum(dmin)


def kernel(z, embedding):
    rows = z.shape[0]
    grid = rows // _BLOCK_ROWS

    zq, idx3, loss_sum = pl.pallas_call(
        _vq_block_kernel,
        grid=(grid,),
        in_specs=[
            pl.BlockSpec((_BLOCK_ROWS, 1024, _CODE_DIM), lambda i: (i, 0, 0)),
            pl.BlockSpec((_NUM_CODES, _CODE_DIM), lambda i: (0, 0)),
        ],
        out_specs=[
            pl.BlockSpec((_BLOCK_ROWS, 1024, _CODE_DIM), lambda i: (i, 0, 0)),
            pl.BlockSpec((1, _BLOCK_ROWS, 1024), lambda i: (i, 0, 0)),
            pl.BlockSpec((1, 1, 1), lambda i: (i, 0, 0),
                         memory_space=pltpu.SMEM),
        ],
        out_shape=[
            jax.ShapeDtypeStruct(z.shape, jnp.float32),
            jax.ShapeDtypeStruct((grid, _BLOCK_ROWS, 1024), jnp.int32),
            jax.ShapeDtypeStruct((grid, 1, 1), jnp.float32),
        ],
        compiler_params=pltpu.CompilerParams(
            dimension_semantics=("parallel",)),
    )(z, embedding)

    indices = idx3.reshape(z.shape[:-1])
    vq_loss = _COMMITMENT_COST * jnp.sum(loss_sum) / (rows * 1024 * _CODE_DIM)
    return (zq, indices, vq_loss)
